# Initial kernel scaffold; baseline (speedup 1.0000x reference)
import functools

import jax
import jax.numpy as jnp
import numpy as np
from jax import lax
from jax.experimental import pallas as pl
from jax.experimental.pallas import tpu as pltpu
from jax.experimental.pallas import tpu_sc as plsc

VOCAB = 1000000
DIM = 64
B = 4096
L = 200
N_SEG = 3

NC = 2
NS = 16
NW = NC * NS
LANES = 16

TOK = B * L
TPW = TOK // NW
CH = 128
NCHUNK = TPW // CH


def _sinusoidal_pe(length, dim):
    pos = np.arange(length)[:, None].astype(np.float64)
    i = np.arange(dim)[None, :]
    angle_rates = 1.0 / np.power(10000.0, (2 * (i // 2)) / np.float64(dim))
    angles = pos * angle_rates
    pe = np.zeros((length, dim), dtype=np.float64)
    pe[:, 0::2] = np.sin(angles[:, 0::2])
    pe[:, 1::2] = np.cos(angles[:, 1::2])
    return pe.astype(np.float32)


_PE = _sinusoidal_pe(L, DIM)


def _combine_kernel(pe_ref, seg_ref, c_ref):
    c_ref[...] = pe_ref[...][:, None, :] + seg_ref[...][None, :, :]


@jax.jit
def _combine(pe, seg_table):
    c3 = pl.pallas_call(
        _combine_kernel,
        out_shape=jax.ShapeDtypeStruct((L, N_SEG, DIM), jnp.float32),
    )(pe, seg_table)
    return c3.reshape(L * N_SEG, DIM)


def _sc_body(seq_hbm, lbl_hbm, tok_hbm, c_hbm, out_hbm,
             idx_v, lbl_v, cidx_v, rows_v, crows_v, sem):
    wid = lax.axis_index("s") * NC + lax.axis_index("c")
    base = wid * TPW

    @pl.loop(0, NCHUNK)
    def chunk(k):
        off = base + k * CH
        pltpu.sync_copy(seq_hbm.at[pl.ds(off, CH)], idx_v)
        pltpu.sync_copy(lbl_hbm.at[pl.ds(off, CH)], lbl_v)

        for g in range(CH // LANES):
            sl = pl.ds(g * LANES, LANES)
            tvec = (off + g * LANES) + lax.iota(jnp.int32, LANES)
            lvec = lax.rem(tvec, jnp.int32(L))
            cidx_v[sl] = lvec * 3 + lbl_v[sl]

        pltpu.async_copy(tok_hbm.at[idx_v], rows_v, sem).wait()
        pltpu.async_copy(c_hbm.at[cidx_v], crows_v, sem).wait()

        @pl.loop(0, CH)
        def add_row(r):
            for d in range(DIM // LANES):
                sl = pl.ds(d * LANES, LANES)
                plsc.addupdate(rows_v.at[r, sl], crows_v[r, sl])

        pltpu.sync_copy(rows_v, out_hbm.at[pl.ds(off, CH)])


@jax.jit
def _sc_embed(seq_flat, lbl_flat, token_table, c_table):
    mesh = plsc.VectorSubcoreMesh(core_axis_name="c", subcore_axis_name="s")
    return pl.kernel(
        _sc_body,
        out_type=jax.ShapeDtypeStruct((TOK, DIM), jnp.float32),
        mesh=mesh,
        scratch_types=[
            pltpu.VMEM((CH,), jnp.int32),
            pltpu.VMEM((CH,), jnp.int32),
            pltpu.VMEM((CH,), jnp.int32),
            pltpu.VMEM((CH, DIM), jnp.float32),
            pltpu.VMEM((CH, DIM), jnp.float32),
            pltpu.SemaphoreType.DMA,
        ],
    )(seq_flat, lbl_flat, token_table, c_table)


def kernel(seq, seg_label, token_table, seg_table):
    pe = jnp.asarray(_PE)
    c_table = _combine(pe, seg_table)
    out = _sc_embed(
        seq.reshape(TOK).astype(jnp.int32),
        seg_label.reshape(TOK).astype(jnp.int32),
        token_table,
        c_table,
    )
    return out.reshape(B, L, DIM)


# speedup vs baseline: 1.8754x; 1.8754x over previous
import functools

import jax
import jax.numpy as jnp
import numpy as np
from jax import lax
from jax.experimental import pallas as pl
from jax.experimental.pallas import tpu as pltpu
from jax.experimental.pallas import tpu_sc as plsc

VOCAB = 1000000
DIM = 64
B = 4096
L = 200
N_SEG = 3

NC = 2
NS = 16
NW = NC * NS
LANES = 16

TOK = B * L
TPW = TOK // NW
CH = 128
NCHUNK = TPW // CH


def _sinusoidal_pe(length, dim):
    pos = np.arange(length)[:, None].astype(np.float64)
    i = np.arange(dim)[None, :]
    angle_rates = 1.0 / np.power(10000.0, (2 * (i // 2)) / np.float64(dim))
    angles = pos * angle_rates
    pe = np.zeros((length, dim), dtype=np.float64)
    pe[:, 0::2] = np.sin(angles[:, 0::2])
    pe[:, 1::2] = np.cos(angles[:, 1::2])
    return pe.astype(np.float32)


_PE = _sinusoidal_pe(L, DIM)


def _combine_kernel(pe_ref, seg_ref, c_ref):
    c_ref[...] = pe_ref[...][:, None, :] + seg_ref[...][None, :, :]


@jax.jit
def _combine(pe, seg_table):
    c3 = pl.pallas_call(
        _combine_kernel,
        out_shape=jax.ShapeDtypeStruct((L, N_SEG, DIM), jnp.float32),
    )(pe, seg_table)
    return c3.reshape(L * N_SEG, DIM)


def _sc_body(seq_hbm, lbl_hbm, tok_hbm, c_hbm, out_hbm,
             idx_v, lbl_v, cidx_v, rows_v, crows_v, sem):
    wid = lax.axis_index("s") * NC + lax.axis_index("c")
    base = wid * TPW

    @pl.loop(0, NCHUNK)
    def chunk(k):
        off = base + k * CH
        pltpu.sync_copy(seq_hbm.at[pl.ds(off, CH)], idx_v)
        pltpu.sync_copy(lbl_hbm.at[pl.ds(off, CH)], lbl_v)

        for g in range(CH // LANES):
            sl = pl.ds(g * LANES, LANES)
            tvec = (off + g * LANES) + lax.iota(jnp.int32, LANES)
            lvec = lax.rem(tvec, jnp.int32(L))
            cidx_v[sl] = lvec * 3 + lbl_v[sl]

        pltpu.async_copy(tok_hbm.at[idx_v], rows_v, sem).wait()
        pltpu.async_copy(c_hbm.at[cidx_v], crows_v, sem).wait()

        @pl.loop(0, CH)
        def add_row(r):
            for d in range(DIM // LANES):
                sl = pl.ds(d * LANES, LANES)
                plsc.addupdate(rows_v.at[r, sl], crows_v[r, sl])

        pltpu.sync_copy(rows_v, out_hbm.at[pl.ds(off, CH)])


@jax.jit
def _sc_embed(seq_flat, lbl_flat, token_table, c_table):
    mesh = plsc.VectorSubcoreMesh(core_axis_name="c", subcore_axis_name="s")
    return pl.kernel(
        _sc_body,
        out_type=jax.ShapeDtypeStruct((TOK, DIM), jnp.float32),
        mesh=mesh,
        compiler_params=pltpu.CompilerParams(use_tc_tiling_on_sc=False),
        scratch_types=[
            pltpu.VMEM((CH,), jnp.int32),
            pltpu.VMEM((CH,), jnp.int32),
            pltpu.VMEM((CH,), jnp.int32),
            pltpu.VMEM((CH, DIM), jnp.float32),
            pltpu.VMEM((CH, DIM), jnp.float32),
            pltpu.SemaphoreType.DMA,
        ],
    )(seq_flat, lbl_flat, token_table, c_table)


def kernel(seq, seg_label, token_table, seg_table):
    pe = jnp.asarray(_PE)
    c_table = _combine(pe, seg_table)
    out = _sc_embed(
        seq.reshape(TOK).astype(jnp.int32),
        seg_label.reshape(TOK).astype(jnp.int32),
        token_table,
        c_table,
    )
    return out.reshape(B, L, DIM)


# speedup vs baseline: 2.3835x; 1.2709x over previous
import functools

import jax
import jax.numpy as jnp
import numpy as np
from jax import lax
from jax.experimental import pallas as pl
from jax.experimental.pallas import tpu as pltpu
from jax.experimental.pallas import tpu_sc as plsc

VOCAB = 1000000
DIM = 64
B = 4096
L = 200
N_SEG = 3

NC = 2
NS = 16
NW = NC * NS
LANES = 16

TOK = B * L
TPW = TOK // NW
CH = 128
NCHUNK = TPW // CH


def _sinusoidal_pe(length, dim):
    pos = np.arange(length)[:, None].astype(np.float64)
    i = np.arange(dim)[None, :]
    angle_rates = 1.0 / np.power(10000.0, (2 * (i // 2)) / np.float64(dim))
    angles = pos * angle_rates
    pe = np.zeros((length, dim), dtype=np.float64)
    pe[:, 0::2] = np.sin(angles[:, 0::2])
    pe[:, 1::2] = np.cos(angles[:, 1::2])
    return pe.astype(np.float32)


_PE = _sinusoidal_pe(L, DIM)


def _combine_kernel(pe_ref, seg_ref, c_ref):
    c_ref[...] = pe_ref[...][:, None, :] + seg_ref[...][None, :, :]


@jax.jit
def _combine(pe, seg_table):
    c3 = pl.pallas_call(
        _combine_kernel,
        out_shape=jax.ShapeDtypeStruct((L, N_SEG, DIM), jnp.float32),
    )(pe, seg_table)
    return c3.reshape(L * N_SEG, DIM)


NBUF = 4


def _sc_body(seq_hbm, lbl_hbm, tok_hbm, c_hbm, out_hbm, *scr):
    idx_v = scr[0:NBUF]
    lbl_v = scr[NBUF:2 * NBUF]
    cidx_v = scr[2 * NBUF:3 * NBUF]
    rows_v = scr[3 * NBUF:4 * NBUF]
    crows_v = scr[4 * NBUF:5 * NBUF]
    fsem = scr[5 * NBUF:6 * NBUF]
    gsem = scr[6 * NBUF:7 * NBUF]
    ssem = scr[7 * NBUF:8 * NBUF]

    wid = lax.axis_index("s") * NC + lax.axis_index("c")
    base = wid * TPW

    @pl.loop(0, NCHUNK, step=NBUF)
    def iteration(k0):
        offs = [base + (k0 + b) * CH for b in range(NBUF)]

        for b in range(NBUF):
            @pl.when(k0 > 0)
            def _drain(b=b):
                pltpu.make_async_copy(
                    rows_v[b], out_hbm.at[pl.ds(offs[b], CH)], ssem[b]).wait()
            pltpu.async_copy(seq_hbm.at[pl.ds(offs[b], CH)], idx_v[b], fsem[b])
            pltpu.async_copy(lbl_hbm.at[pl.ds(offs[b], CH)], lbl_v[b], fsem[b])

        for b in range(NBUF):
            pltpu.make_async_copy(
                seq_hbm.at[pl.ds(offs[b], CH)], idx_v[b], fsem[b]).wait()
            pltpu.make_async_copy(
                lbl_hbm.at[pl.ds(offs[b], CH)], lbl_v[b], fsem[b]).wait()
            for g in range(CH // LANES):
                sl = pl.ds(g * LANES, LANES)
                tvec = (offs[b] + g * LANES) + lax.iota(jnp.int32, LANES)
                lvec = lax.rem(tvec, jnp.int32(L))
                cidx_v[b][sl] = lvec * 3 + lbl_v[b][sl]
            pltpu.async_copy(tok_hbm.at[idx_v[b]], rows_v[b], gsem[b])
            pltpu.async_copy(c_hbm.at[cidx_v[b]], crows_v[b], gsem[b])

        for b in range(NBUF):
            pltpu.make_async_copy(
                tok_hbm.at[idx_v[b]], rows_v[b], gsem[b]).wait()
            pltpu.make_async_copy(
                c_hbm.at[cidx_v[b]], crows_v[b], gsem[b]).wait()

            @pl.loop(0, CH, unroll=8)
            def add_row(r, b=b):
                for d in range(DIM // LANES):
                    sl = pl.ds(d * LANES, LANES)
                    plsc.addupdate(rows_v[b].at[r, sl], crows_v[b][r, sl])

            pltpu.async_copy(rows_v[b], out_hbm.at[pl.ds(offs[b], CH)], ssem[b])

    for b in range(NBUF):
        off = base + (NCHUNK - NBUF + b) * CH
        pltpu.make_async_copy(
            rows_v[b], out_hbm.at[pl.ds(off, CH)], ssem[b]).wait()


@jax.jit
def _sc_embed(seq_flat, lbl_flat, token_table, c_table):
    mesh = plsc.VectorSubcoreMesh(core_axis_name="c", subcore_axis_name="s")
    return pl.kernel(
        _sc_body,
        out_type=jax.ShapeDtypeStruct((TOK, DIM), jnp.float32),
        mesh=mesh,
        compiler_params=pltpu.CompilerParams(use_tc_tiling_on_sc=False),
        scratch_types=(
            [pltpu.VMEM((CH,), jnp.int32)] * NBUF
            + [pltpu.VMEM((CH,), jnp.int32)] * NBUF
            + [pltpu.VMEM((CH,), jnp.int32)] * NBUF
            + [pltpu.VMEM((CH, DIM), jnp.float32)] * NBUF
            + [pltpu.VMEM((CH, DIM), jnp.float32)] * NBUF
            + [pltpu.SemaphoreType.DMA] * (3 * NBUF)
        ),
    )(seq_flat, lbl_flat, token_table, c_table)


def kernel(seq, seg_label, token_table, seg_table):
    pe = jnp.asarray(_PE)
    c_table = _combine(pe, seg_table)
    out = _sc_embed(
        seq.reshape(TOK).astype(jnp.int32),
        seg_label.reshape(TOK).astype(jnp.int32),
        token_table,
        c_table,
    )
    return out.reshape(B, L, DIM)
